# baseline (speedup 1.0000x reference)
import functools

import jax
import jax.numpy as jnp
from jax import lax
from jax.experimental import pallas as pl
from jax.experimental.pallas import tpu as pltpu
from jax.experimental.pallas import tpu_sc as plsc

_NC = 2
_NS = 16
_NW = _NC * _NS


def _emb_lookup(x2d, table, *, b_per_w, chunk, nbuf):
    d = table.shape[1]
    n_rows, n_cols = x2d.shape
    w_per_row = n_cols // b_per_w
    n_chunks = b_per_w // chunk
    mesh = plsc.VectorSubcoreMesh(core_axis_name="c", subcore_axis_name="s")

    @functools.partial(
        pl.kernel,
        mesh=mesh,
        out_type=jax.ShapeDtypeStruct((n_rows, n_cols, d), jnp.float32),
        scratch_types=(
            [pltpu.VMEM((b_per_w,), jnp.int32)]
            + [pltpu.VMEM((chunk, d), jnp.float32) for _ in range(nbuf)]
            + [pltpu.SemaphoreType.DMA, pltpu.SemaphoreType.DMA]
        ),
    )
    def body(idx_hbm, table_hbm, out_hbm, idx_v, *rest):
        bufs = rest[:nbuf]
        g_sem, w_sem = rest[nbuf], rest[nbuf + 1]
        wid = lax.axis_index("s") * _NC + lax.axis_index("c")
        row = wid // w_per_row
        col0 = (wid % w_per_row) * b_per_w
        pltpu.sync_copy(idx_hbm.at[row, pl.ds(col0, b_per_w)], idx_v)

        def gather(c):
            return pltpu.async_copy(
                table_hbm.at[idx_v.at[pl.ds(c * chunk, chunk)]],
                bufs[c % nbuf], g_sem)

        def writeback(c):
            return pltpu.async_copy(
                bufs[c % nbuf],
                out_hbm.at[row, pl.ds(col0 + c * chunk, chunk)], w_sem)

        gathers = [gather(c) for c in range(min(nbuf - 1, n_chunks))]
        writes = []
        for c in range(n_chunks):
            gathers[c].wait()
            writes.append(writeback(c))
            nxt = c + nbuf - 1
            if nxt < n_chunks:
                if nxt >= nbuf:
                    writes[nxt - nbuf].wait()
                gathers.append(gather(nxt))
        for c in range(max(0, n_chunks - nbuf), n_chunks):
            writes[c].wait()

    return body(x2d, table)


def kernel(x, table):
    b, s = x.shape
    return _emb_lookup(x.astype(jnp.int32), table,
                       b_per_w=(b * s) // _NW, chunk=8, nbuf=8)


# speedup vs baseline: 1.0281x; 1.0281x over previous
import functools

import jax
import jax.numpy as jnp
from jax import lax
from jax.experimental import pallas as pl
from jax.experimental.pallas import tpu as pltpu
from jax.experimental.pallas import tpu_sc as plsc

_NC = 2
_NS = 16
_NW = _NC * _NS


def _emb_lookup(x2d, table, *, b_per_w, chunk, nbuf):
    d = table.shape[1]
    n_rows, n_cols = x2d.shape
    w_per_row = n_cols // b_per_w
    sizes = [8, 24] + [chunk] * ((b_per_w - 64) // chunk) + [24, 8]
    assert sum(sizes) == b_per_w
    offs = [sum(sizes[:i]) for i in range(len(sizes))]
    n_chunks = len(sizes)
    mesh = plsc.VectorSubcoreMesh(core_axis_name="c", subcore_axis_name="s")

    @functools.partial(
        pl.kernel,
        mesh=mesh,
        out_type=jax.ShapeDtypeStruct((n_rows, n_cols, d), jnp.float32),
        scratch_types=(
            [pltpu.VMEM((b_per_w,), jnp.int32)]
            + [pltpu.VMEM((chunk, d), jnp.float32) for _ in range(nbuf)]
            + [pltpu.SemaphoreType.DMA, pltpu.SemaphoreType.DMA]
        ),
    )
    def body(idx_hbm, table_hbm, out_hbm, idx_v, *rest):
        bufs = rest[:nbuf]
        g_sem, w_sem = rest[nbuf], rest[nbuf + 1]
        wid = lax.axis_index("s") * _NC + lax.axis_index("c")
        row = wid // w_per_row
        col0 = (wid % w_per_row) * b_per_w
        pltpu.sync_copy(idx_hbm.at[row, pl.ds(col0, b_per_w)], idx_v)

        def gather(c):
            return pltpu.async_copy(
                table_hbm.at[idx_v.at[pl.ds(offs[c], sizes[c])]],
                bufs[c % nbuf].at[pl.ds(0, sizes[c])], g_sem)

        def writeback(c):
            return pltpu.async_copy(
                bufs[c % nbuf].at[pl.ds(0, sizes[c])],
                out_hbm.at[row, pl.ds(col0 + offs[c], sizes[c])], w_sem)

        gathers = [gather(c) for c in range(min(nbuf - 1, n_chunks))]
        writes = []
        for c in range(n_chunks):
            gathers[c].wait()
            writes.append(writeback(c))
            nxt = c + nbuf - 1
            if nxt < n_chunks:
                if nxt >= nbuf:
                    writes[nxt - nbuf].wait()
                gathers.append(gather(nxt))
        for c in range(max(0, n_chunks - nbuf), n_chunks):
            writes[c].wait()

    return body(x2d, table)


def kernel(x, table):
    b, s = x.shape
    return _emb_lookup(x.astype(jnp.int32), table,
                       b_per_w=(b * s) // _NW, chunk=32, nbuf=3)
